# baseline (speedup 1.0000x reference)
import functools

import jax
import jax.numpy as jnp
from jax import lax
from jax.experimental import pallas as pl
from jax.experimental.pallas import tpu as pltpu
from jax.experimental.pallas import tpu_sc as plsc

N_MOD = 2
DIM = 2048
HID = 5632
T = 8192

RB = 1024
Tp = T + RB
NB = Tp // RB
HB = 256
NH = HID // HB

NC, NS = 2, 16
NW = NC * NS
RPW = Tp // NW
CHUNK = 24
NCH = RPW // CHUNK


@functools.cache
def _sc_kernels():
    mesh = plsc.VectorSubcoreMesh(core_axis_name="c", subcore_axis_name="s",
                                  num_cores=NC, num_subcores=NS)

    @functools.partial(
        pl.kernel,
        out_type=jax.ShapeDtypeStruct((Tp, DIM), jnp.float32),
        mesh=mesh,
        scratch_types=[
            pltpu.VMEM((RPW,), jnp.int32),
            pltpu.VMEM((CHUNK, DIM), jnp.float32),
            pltpu.SemaphoreType.DMA,
        ],
    )
    def _sc_gather(x_hbm, gidx_hbm, xs_hbm, idx_v, rows_v, sem):
        wid = lax.axis_index("s") * NC + lax.axis_index("c")
        base = wid * RPW
        pltpu.sync_copy(gidx_hbm.at[pl.ds(base, RPW)], idx_v)
        for c in range(NCH):
            pltpu.async_copy(
                x_hbm.at[idx_v.at[pl.ds(c * CHUNK, CHUNK)]], rows_v, sem
            ).wait()
            pltpu.sync_copy(rows_v, xs_hbm.at[pl.ds(base + c * CHUNK, CHUNK)])

    @functools.partial(
        pl.kernel,
        out_type=jax.ShapeDtypeStruct((T + RB, DIM), jnp.float32),
        mesh=mesh,
        scratch_types=[
            pltpu.VMEM((NCH, CHUNK), jnp.int32),
            pltpu.VMEM((CHUNK, DIM), jnp.float32),
            pltpu.SemaphoreType.DMA,
        ],
    )
    def _sc_scatter(y_hbm, sidx_hbm, out_hbm, sidx_v, rows_v, sem):
        wid = lax.axis_index("s") * NC + lax.axis_index("c")
        base = wid * RPW
        pltpu.sync_copy(sidx_hbm.at[wid], sidx_v)
        for c in range(NCH):
            pltpu.sync_copy(y_hbm.at[pl.ds(base + c * CHUNK, CHUNK)], rows_v)
            pltpu.async_copy(rows_v, out_hbm.at[sidx_v.at[c]], sem).wait()

    return _sc_gather, _sc_scatter


def _ffn_body(eid_ref, xs_ref, w1_ref, w3_ref, w2_ref, nw_ref, out_ref, acc_ref):
    h = pl.program_id(1)

    @pl.when(h == 0)
    def _():
        acc_ref[...] = jnp.zeros_like(acc_ref)

    x = xs_ref[...].astype(jnp.bfloat16)
    w1b = lax.bitcast_convert_type(w1_ref[0], jnp.bfloat16)
    w3b = lax.bitcast_convert_type(w3_ref[0], jnp.bfloat16)
    w2b = lax.bitcast_convert_type(w2_ref[0], jnp.bfloat16)
    x1 = lax.dot_general(x, w1b, (((1,), (1,)), ((), ())),
                         preferred_element_type=jnp.float32)
    x3 = lax.dot_general(x, w3b, (((1,), (1,)), ((), ())),
                         preferred_element_type=jnp.float32)
    hidden = ((x1 * jax.nn.sigmoid(x1)) * x3).astype(jnp.bfloat16)
    acc_ref[...] += lax.dot_general(hidden, w2b, (((1,), (0,)), ((), ())),
                                    preferred_element_type=jnp.float32)

    @pl.when(h == NH - 1)
    def _():
        a = acc_ref[...]
        ms = jnp.mean(a * a, axis=-1, keepdims=True)
        out_ref[...] = a * lax.rsqrt(ms + 1e-5) * nw_ref[0]


_ffn_call = pl.pallas_call(
    _ffn_body,
    grid_spec=pltpu.PrefetchScalarGridSpec(
        num_scalar_prefetch=1,
        grid=(NB, NH),
        in_specs=[
            pl.BlockSpec((RB, DIM), lambda i, h, eid: (i, 0)),
            pl.BlockSpec((1, HB, DIM), lambda i, h, eid: (eid[i], h, 0)),
            pl.BlockSpec((1, HB, DIM), lambda i, h, eid: (eid[i], h, 0)),
            pl.BlockSpec((1, HB, DIM), lambda i, h, eid: (eid[i], h, 0)),
            pl.BlockSpec((1, 1, DIM), lambda i, h, eid: (eid[i], 0, 0)),
        ],
        out_specs=pl.BlockSpec((RB, DIM), lambda i, h, eid: (i, 0)),
        scratch_shapes=[pltpu.VMEM((RB, DIM), jnp.float32)],
    ),
    out_shape=jax.ShapeDtypeStruct((Tp, DIM), jnp.float32),
    compiler_params=pltpu.CompilerParams(
        dimension_semantics=("arbitrary", "arbitrary")),
)


def _routing_indices(modality_masks):
    m0 = modality_masks[0]
    m0i = m0.astype(jnp.int32)
    c0 = jnp.sum(m0i)
    nb0 = (c0 + RB - 1) // RB
    start1 = nb0 * RB
    pos0 = jnp.cumsum(m0i) - 1
    pos1 = start1 + jnp.cumsum(1 - m0i) - 1
    dst = jnp.where(m0, pos0, pos1)
    gidx = (T + (jnp.arange(Tp, dtype=jnp.int32) % RB)).at[dst].set(
        jnp.arange(T, dtype=jnp.int32))
    eid = (jnp.arange(NB, dtype=jnp.int32) >= nb0).astype(jnp.int32)
    return gidx, eid


def kernel(x, modality_masks, w1, w3, w2, norm_w):
    gidx, eid = _routing_indices(modality_masks)
    gather_idx = jnp.where(gidx >= T, 0, gidx)
    scatter_idx = gidx.reshape(NW, NCH, CHUNK)

    sc_gather, sc_scatter = _sc_kernels()
    xs = sc_gather(x, gather_idx)
    w1b = lax.bitcast_convert_type(w1.astype(jnp.bfloat16), jnp.int16)
    w3b = lax.bitcast_convert_type(w3.astype(jnp.bfloat16), jnp.int16)
    w2tb = lax.bitcast_convert_type(
        jnp.swapaxes(w2, 1, 2).astype(jnp.bfloat16), jnp.int16)
    y = _ffn_call(eid, xs, w1b, w3b, w2tb, norm_w.reshape(N_MOD, 1, DIM))
    out = sc_scatter(y, scatter_idx)
    return out[:T]


# speedup vs baseline: 1.1639x; 1.1639x over previous
import functools

import jax
import jax.numpy as jnp
from jax import lax
from jax.experimental import pallas as pl
from jax.experimental.pallas import tpu as pltpu
from jax.experimental.pallas import tpu_sc as plsc

N_MOD = 2
DIM = 2048
HID = 5632
T = 8192

RB = 1024
Tp = T + RB
NB = Tp // RB
HB = 256
NH = HID // HB

NC, NS = 2, 16
NW = NC * NS
RPW = Tp // NW
CHUNK = 24
NCH = RPW // CHUNK


@functools.cache
def _sc_kernels():
    mesh = plsc.VectorSubcoreMesh(core_axis_name="c", subcore_axis_name="s",
                                  num_cores=NC, num_subcores=NS)

    @functools.partial(
        pl.kernel,
        out_type=jax.ShapeDtypeStruct((Tp, DIM), jnp.float32),
        mesh=mesh,
        scratch_types=[
            pltpu.VMEM((RPW,), jnp.int32),
            pltpu.VMEM((CHUNK, DIM), jnp.float32),
            pltpu.VMEM((CHUNK, DIM), jnp.float32),
            pltpu.SemaphoreType.DMA,
            pltpu.SemaphoreType.DMA,
        ],
    )
    def _sc_gather(x_hbm, gidx_hbm, xs_hbm, idx_v, rows0, rows1, sem0, sem1):
        wid = lax.axis_index("s") * NC + lax.axis_index("c")
        base = wid * RPW
        pltpu.sync_copy(gidx_hbm.at[pl.ds(base, RPW)], idx_v)
        bufs, sems = (rows0, rows1), (sem0, sem1)
        copies = [None] * NCH
        copies[0] = pltpu.async_copy(
            x_hbm.at[idx_v.at[pl.ds(0, CHUNK)]], bufs[0], sems[0])
        for c in range(NCH):
            if c + 1 < NCH:
                copies[c + 1] = pltpu.async_copy(
                    x_hbm.at[idx_v.at[pl.ds((c + 1) * CHUNK, CHUNK)]],
                    bufs[(c + 1) % 2], sems[(c + 1) % 2])
            copies[c].wait()
            pltpu.sync_copy(bufs[c % 2],
                            xs_hbm.at[pl.ds(base + c * CHUNK, CHUNK)])

    @functools.partial(
        pl.kernel,
        out_type=jax.ShapeDtypeStruct((T, DIM), jnp.float32),
        mesh=mesh,
        scratch_types=[
            pltpu.VMEM((NCH, CHUNK), jnp.int32),
            pltpu.VMEM((CHUNK, DIM), jnp.float32),
            pltpu.VMEM((CHUNK, DIM), jnp.float32),
            pltpu.SemaphoreType.DMA,
            pltpu.SemaphoreType.DMA,
        ],
    )
    def _sc_scatter(y_hbm, sidx_hbm, out_hbm, sidx_v, rows0, rows1, sem0, sem1):
        wid = lax.axis_index("s") * NC + lax.axis_index("c")
        base = wid * RPW
        pltpu.sync_copy(sidx_hbm.at[wid], sidx_v)
        bufs = (rows0, rows1)
        sems = (sem0, sem1)
        scat = [None] * NCH
        pltpu.sync_copy(y_hbm.at[pl.ds(base, CHUNK)], bufs[0])
        for c in range(NCH):
            scat[c] = pltpu.async_copy(
                bufs[c % 2], out_hbm.at[sidx_v.at[c]], sems[c % 2])
            if c + 1 < NCH:
                if c >= 1:
                    scat[c - 1].wait()
                pltpu.sync_copy(
                    y_hbm.at[pl.ds(base + (c + 1) * CHUNK, CHUNK)],
                    bufs[(c + 1) % 2])
        if NCH >= 2:
            scat[NCH - 2].wait()
        scat[NCH - 1].wait()

    return _sc_gather, _sc_scatter


def _ffn_body(eid_ref, xs_ref, w1_ref, w3_ref, w2_ref, nw_ref, out_ref):
    h = pl.program_id(1)

    @pl.when(h == 0)
    def _():
        out_ref[...] = jnp.zeros_like(out_ref)

    x = xs_ref[...]
    x1 = lax.dot_general(x, w1_ref[0], (((1,), (1,)), ((), ())),
                         preferred_element_type=jnp.float32)
    x3 = lax.dot_general(x, w3_ref[0], (((1,), (1,)), ((), ())),
                         preferred_element_type=jnp.float32)
    hidden = (x1 * jax.nn.sigmoid(x1)) * x3
    out_ref[...] += lax.dot_general(hidden, w2_ref[0], (((1,), (1,)), ((), ())),
                                    preferred_element_type=jnp.float32)

    @pl.when(h == NH - 1)
    def _():
        a = out_ref[...]
        ms = jnp.mean(a * a, axis=-1, keepdims=True)
        out_ref[...] = a * lax.rsqrt(ms + 1e-5) * nw_ref[0]


_ffn_call = pl.pallas_call(
    _ffn_body,
    grid_spec=pltpu.PrefetchScalarGridSpec(
        num_scalar_prefetch=1,
        grid=(NB, NH),
        in_specs=[
            pl.BlockSpec((RB, DIM), lambda i, h, eid: (i, 0)),
            pl.BlockSpec((1, HB, DIM), lambda i, h, eid: (eid[i], h, 0)),
            pl.BlockSpec((1, HB, DIM), lambda i, h, eid: (eid[i], h, 0)),
            pl.BlockSpec((1, DIM, HB), lambda i, h, eid: (eid[i], 0, h)),
            pl.BlockSpec((1, 1, DIM), lambda i, h, eid: (eid[i], 0, 0)),
        ],
        out_specs=pl.BlockSpec((RB, DIM), lambda i, h, eid: (i, 0)),
    ),
    out_shape=jax.ShapeDtypeStruct((Tp, DIM), jnp.float32),
    compiler_params=pltpu.CompilerParams(
        dimension_semantics=("arbitrary", "arbitrary")),
)


def _routing_indices(modality_masks):
    m0 = modality_masks[0]
    m0i = m0.astype(jnp.int32)
    c0 = jnp.sum(m0i)
    c1 = T - c0
    nb0 = (c0 + RB - 1) // RB
    start1 = nb0 * RB
    pos0 = jnp.cumsum(m0i) - 1
    pos1 = start1 + jnp.cumsum(1 - m0i) - 1
    dst = jnp.where(m0, pos0, pos1)
    t0 = jnp.argmax(m0).astype(jnp.int32)
    t1 = jnp.argmax(~m0).astype(jnp.int32)
    t0e = jnp.where(c0 > 0, t0, t1)
    t1e = jnp.where(c1 > 0, t1, t0)
    init = jnp.where(jnp.arange(Tp, dtype=jnp.int32) < start1, t0e, t1e)
    gidx = init.at[dst].set(jnp.arange(T, dtype=jnp.int32))
    eid = ((jnp.arange(NB) >= nb0) & (c1 > 0)).astype(jnp.int32)
    return gidx, eid


def kernel(x, modality_masks, w1, w3, w2, norm_w):
    gidx, eid = _routing_indices(modality_masks)
    scatter_idx = gidx.reshape(NW, NCH, CHUNK)

    sc_gather, sc_scatter = _sc_kernels()
    xs = sc_gather(x, gidx)
    y = _ffn_call(eid, xs, w1, w3, w2, norm_w.reshape(N_MOD, 1, DIM))
    out = sc_scatter(y, scatter_idx)
    return out
